# baseline (speedup 1.0000x reference)
import functools

import jax
import jax.numpy as jnp
from jax import lax
from jax.experimental import pallas as pl
from jax.experimental.pallas import tpu as pltpu
from jax.experimental.pallas import tpu_sc as plsc

N = 10000
E = 160000
D = 256
H = 128
NC = 2
NS = 16
CH = 128

EP = 1280 * CH
EROWS = EP // CH
TR_A = EROWS // 32
TR_C = EROWS // NS
NPAD = 10240

_MESH = plsc.VectorSubcoreMesh(core_axis_name="c", subcore_axis_name="s")


@functools.partial(
    pl.kernel,
    out_type=jax.ShapeDtypeStruct((NC, NPAD), jnp.float32),
    mesh=_MESH,
    scratch_types=[
        pltpu.VMEM((TR_A, CH), jnp.int32),
        pltpu.VMEM((TR_A, CH), jnp.float32),
        pltpu.VMEM((640,), jnp.float32),
        pltpu.VMEM_SHARED((NPAD,), jnp.float32),
    ],
)
def _rowsum(rows_hbm, ev_hbm, out_hbm, rows_v, ev_v, zbuf, acc):
    c = lax.axis_index("c")
    s = lax.axis_index("s")
    t = c * NS + s
    pltpu.sync_copy(rows_hbm.at[pl.ds(t * TR_A, TR_A)], rows_v)
    pltpu.sync_copy(ev_hbm.at[pl.ds(t * TR_A, TR_A)], ev_v)
    z16 = jnp.zeros((16,), jnp.float32)
    for j in range(40):
        zbuf[pl.ds(j * 16, 16)] = z16
    pltpu.sync_copy(zbuf, acc.at[pl.ds(s * 640, 640)])
    plsc.subcore_barrier()

    def body(g, carry):
        pltpu.sync_copy(ev_v.at[g], acc.at[rows_v.at[g]], add=True)
        return carry

    lax.fori_loop(0, TR_A, body, 0)
    plsc.subcore_barrier()
    pltpu.sync_copy(acc.at[pl.ds(s * 640, 640)], out_hbm.at[c, pl.ds(s * 640, 640)])


def _dis_body(p_ref, dis_ref):
    rs = p_ref[0] + p_ref[1]
    dis_ref[...] = jnp.where(rs > 0.0, lax.rsqrt(rs), 0.0)


_dis = pl.pallas_call(
    _dis_body,
    out_shape=jax.ShapeDtypeStruct((NPAD,), jnp.float32),
)


def _mm_body(x_ref, w_ref, dis_ref, ma_ref, mb_ref):
    m = jnp.dot(x_ref[...], w_ref[...], preferred_element_type=jnp.float32)
    md = m * dis_ref[...]
    ma_ref[...] = md[:, :H]
    mb_ref[...] = md[:, H:]


_matmul = pl.pallas_call(
    _mm_body,
    grid=(10,),
    in_specs=[
        pl.BlockSpec((N // 10, D), lambda i: (i, 0)),
        pl.BlockSpec((D, D), lambda i: (0, 0)),
        pl.BlockSpec((N // 10, 1), lambda i: (i, 0)),
    ],
    out_specs=[
        pl.BlockSpec((N // 10, H), lambda i: (i, 0)),
        pl.BlockSpec((N // 10, H), lambda i: (i, 0)),
    ],
    out_shape=[
        jax.ShapeDtypeStruct((N, H), jnp.float32),
        jax.ShapeDtypeStruct((N, H), jnp.float32),
    ],
)


@functools.partial(
    pl.kernel,
    out_type=jax.ShapeDtypeStruct((N, D), jnp.float32),
    mesh=_MESH,
    scratch_types=[
        pltpu.VMEM((TR_C, CH), jnp.int32),
        pltpu.VMEM((TR_C, CH), jnp.int32),
        pltpu.VMEM((TR_C, CH), jnp.float32),
        pltpu.VMEM((CH // 2, H), jnp.float32),
        pltpu.VMEM((CH // 2, H), jnp.float32),
        pltpu.VMEM_SHARED((NPAD, H), jnp.float32),
        pltpu.SemaphoreType.DMA,
        pltpu.SemaphoreType.DMA,
        pltpu.SemaphoreType.DMA,
        pltpu.SemaphoreType.DMA,
    ],
)
def _spmm(rows_hbm, cols_hbm, ev_hbm, ma_hbm, mb_hbm, out_hbm,
          rows_v, cols_v, ev_v, g0, g1, acc, sg0, sg1, ss0, ss1):
    c = lax.axis_index("c")
    s = lax.axis_index("s")
    HC = CH // 2
    pltpu.sync_copy(rows_hbm.at[pl.ds(s * TR_C, TR_C)], rows_v)
    pltpu.sync_copy(cols_hbm.at[pl.ds(s * TR_C, TR_C)], cols_v)
    pltpu.sync_copy(ev_hbm.at[pl.ds(s * TR_C, TR_C)], ev_v)

    z16 = jnp.zeros((16,), jnp.float32)

    def zrow(i, carry):
        for j in range(H // 16):
            g0[i, pl.ds(j * 16, 16)] = z16
        return carry

    lax.fori_loop(0, HC, zrow, 0)
    for k in range(640 // HC):
        pltpu.sync_copy(g0, acc.at[pl.ds(s * 640 + k * HC, HC)])
    plsc.subcore_barrier()

    def start_gather(i, b, buf, sem):
        idx = cols_v.at[i, pl.ds(b * HC, HC)]

        @pl.when(c == 0)
        def _():
            pltpu.async_copy(ma_hbm.at[idx], buf, sem)

        @pl.when(c == 1)
        def _():
            pltpu.async_copy(mb_hbm.at[idx], buf, sem)

    def wait_gather(i, b, buf, sem):
        idx = cols_v.at[i, pl.ds(b * HC, HC)]

        @pl.when(c == 0)
        def _():
            pltpu.make_async_copy(ma_hbm.at[idx], buf, sem).wait()

        @pl.when(c == 1)
        def _():
            pltpu.make_async_copy(mb_hbm.at[idx], buf, sem).wait()

    dnums = lax.GatherDimensionNumbers(
        offset_dims=(), collapsed_slice_dims=(0,), start_index_map=(0,))

    def scale(i, b, buf):
        def kk_body(kk, carry2):
            ev16 = ev_v[i, pl.ds(b * HC + kk * 16, 16)]
            for j in range(16):
                sval = lax.gather(
                    ev16, jnp.full((16, 1), j, jnp.int32), dnums, (1,),
                    mode=lax.GatherScatterMode.PROMISE_IN_BOUNDS)
                e = kk * 16 + j
                for h in range(H // 16):
                    buf[e, pl.ds(h * 16, 16)] = (
                        buf[e, pl.ds(h * 16, 16)] * sval)
            return carry2

        lax.fori_loop(0, HC // 16, kk_body, 0)

    def start_scatter(i, b, buf, sem):
        pltpu.async_copy(buf, acc.at[rows_v.at[i, pl.ds(b * HC, HC)]],
                         sem, add=True)

    def wait_scatter(i, b, buf, sem):
        pltpu.make_async_copy(
            buf, acc.at[rows_v.at[i, pl.ds(b * HC, HC)]], sem).wait()

    start_gather(0, 0, g0, sg0)

    def iter_body(i, carry):
        wait_gather(i, 0, g0, sg0)

        @pl.when(i > 0)
        def _():
            wait_scatter(i - 1, 1, g1, ss1)

        start_gather(i, 1, g1, sg1)
        scale(i, 0, g0)
        start_scatter(i, 0, g0, ss0)
        wait_gather(i, 1, g1, sg1)
        scale(i, 1, g1)
        wait_scatter(i, 0, g0, ss0)

        @pl.when(i + 1 < TR_C)
        def _():
            start_gather(i + 1, 0, g0, sg0)

        start_scatter(i, 1, g1, ss1)
        return carry

    lax.fori_loop(0, TR_C, iter_body, 0)
    wait_scatter(TR_C - 1, 1, g1, ss1)
    plsc.subcore_barrier()

    for k in range(5):
        r0 = s * 640 + k * CH

        @pl.when(r0 + CH <= N)
        def _():
            pltpu.sync_copy(acc.at[pl.ds(r0, CH)],
                            out_hbm.at[pl.ds(r0, CH), pl.ds(c * H, H)])

    @pl.when(s == NS - 1)
    def _():
        r0t = (NS - 1) * 640 + 3 * CH
        pltpu.sync_copy(acc.at[pl.ds(r0t, N - r0t)],
                        out_hbm.at[pl.ds(r0t, N - r0t), pl.ds(c * H, H)])


def _scale_body(a_ref, dis_ref, o_ref):
    o_ref[...] = a_ref[...] * dis_ref[...]


_scale = pl.pallas_call(
    _scale_body,
    grid=(10,),
    in_specs=[
        pl.BlockSpec((N // 10, D), lambda i: (i, 0)),
        pl.BlockSpec((N // 10, 1), lambda i: (i, 0)),
    ],
    out_specs=pl.BlockSpec((N // 10, D), lambda i: (i, 0)),
    out_shape=jax.ShapeDtypeStruct((N, D), jnp.float32),
)


def kernel(x, edge_index, edge_values, W):
    row = edge_index[0].astype(jnp.int32)
    col = edge_index[1].astype(jnp.int32)
    ev = edge_values.astype(jnp.float32)
    rows2d = jnp.pad(row, (0, EP - E)).reshape(EROWS, CH)
    cols2d = jnp.pad(col, (0, EP - E)).reshape(EROWS, CH)
    ev2d = jnp.pad(ev, (0, EP - E)).reshape(EROWS, CH)

    partials = _rowsum(rows2d, ev2d)
    dis = _dis(partials)
    disN = dis[:N].reshape(N, 1)
    ma, mb = _matmul(x, W, disN)
    raw = _spmm(rows2d, cols2d, ev2d, ma, mb)
    return _scale(raw, disN)


# speedup vs baseline: 1.1792x; 1.1792x over previous
import functools

import jax
import jax.numpy as jnp
from jax import lax
from jax.experimental import pallas as pl
from jax.experimental.pallas import tpu as pltpu
from jax.experimental.pallas import tpu_sc as plsc

N = 10000
E = 160000
D = 256
H = 128
NC = 2
NS = 16
CH = 128

EP = 1280 * CH
EROWS = EP // CH
TR_A = EROWS // 32
TR_C = EROWS // NS
BLK_C = TR_C // 2
NPAD = 10240

_MESH = plsc.VectorSubcoreMesh(core_axis_name="c", subcore_axis_name="s")


@functools.partial(
    pl.kernel,
    out_type=jax.ShapeDtypeStruct((NC, NPAD), jnp.float32),
    mesh=_MESH,
    scratch_types=[
        pltpu.VMEM((TR_A, CH), jnp.int32),
        pltpu.VMEM((TR_A, CH), jnp.float32),
        pltpu.VMEM((640,), jnp.float32),
        pltpu.VMEM_SHARED((NPAD,), jnp.float32),
    ],
)
def _rowsum(rows_hbm, ev_hbm, out_hbm, rows_v, ev_v, zbuf, acc):
    c = lax.axis_index("c")
    s = lax.axis_index("s")
    t = c * NS + s
    pltpu.sync_copy(rows_hbm.at[pl.ds(t * TR_A, TR_A)], rows_v)
    pltpu.sync_copy(ev_hbm.at[pl.ds(t * TR_A, TR_A)], ev_v)
    z16 = jnp.zeros((16,), jnp.float32)
    for j in range(40):
        zbuf[pl.ds(j * 16, 16)] = z16
    pltpu.sync_copy(zbuf, acc.at[pl.ds(s * 640, 640)])
    plsc.subcore_barrier()

    def body(g, carry):
        pltpu.sync_copy(ev_v.at[g], acc.at[rows_v.at[g]], add=True)
        return carry

    lax.fori_loop(0, TR_A, body, 0)
    plsc.subcore_barrier()
    pltpu.sync_copy(acc.at[pl.ds(s * 640, 640)], out_hbm.at[c, pl.ds(s * 640, 640)])


def _dis_body(p_ref, dis_ref):
    rs = p_ref[0] + p_ref[1]
    dis_ref[...] = jnp.where(rs > 0.0, lax.rsqrt(rs), 0.0)


_dis = pl.pallas_call(
    _dis_body,
    out_shape=jax.ShapeDtypeStruct((NPAD,), jnp.float32),
)


def _mm_body(x_ref, w_ref, dis_ref, ma_ref, mb_ref):
    m = jnp.dot(x_ref[...], w_ref[...], preferred_element_type=jnp.float32)
    md = m * dis_ref[...]
    ma_ref[...] = md[:, :H]
    mb_ref[...] = md[:, H:]


_matmul = pl.pallas_call(
    _mm_body,
    grid=(10,),
    in_specs=[
        pl.BlockSpec((N // 10, D), lambda i: (i, 0)),
        pl.BlockSpec((D, D), lambda i: (0, 0)),
        pl.BlockSpec((N // 10, 1), lambda i: (i, 0)),
    ],
    out_specs=[
        pl.BlockSpec((N // 10, H), lambda i: (i, 0)),
        pl.BlockSpec((N // 10, H), lambda i: (i, 0)),
    ],
    out_shape=[
        jax.ShapeDtypeStruct((N, H), jnp.float32),
        jax.ShapeDtypeStruct((N, H), jnp.float32),
    ],
)


@functools.partial(
    pl.kernel,
    out_type=jax.ShapeDtypeStruct((N, D), jnp.float32),
    mesh=_MESH,
    scratch_types=[
        pltpu.VMEM((BLK_C, CH), jnp.int32),
        pltpu.VMEM((BLK_C, CH), jnp.int32),
        pltpu.VMEM((BLK_C, CH), jnp.float32),
        pltpu.VMEM((CH // 2, H), jnp.float32),
        pltpu.VMEM((CH // 2, H), jnp.float32),
        pltpu.VMEM((CH // 2, H), jnp.float32),
        pltpu.VMEM_SHARED((NPAD, H), jnp.float32),
        pltpu.SemaphoreType.DMA,
        pltpu.SemaphoreType.DMA,
        pltpu.SemaphoreType.DMA,
        pltpu.SemaphoreType.DMA,
        pltpu.SemaphoreType.DMA,
        pltpu.SemaphoreType.DMA,
    ],
)
def _spmm(rows_hbm, cols_hbm, ev_hbm, ma_hbm, mb_hbm, out_hbm,
          rows_v, cols_v, ev_v, g0, g1, g2, acc,
          sg0, sg1, sg2, ss0, ss1, ss2):
    c = lax.axis_index("c")
    s = lax.axis_index("s")
    HC = CH // 2
    GB = (g0, g1, g2)
    SG = (sg0, sg1, sg2)
    SS = (ss0, ss1, ss2)

    z16 = jnp.zeros((16,), jnp.float32)

    def zrow(i, carry):
        for j in range(H // 16):
            g0[i, pl.ds(j * 16, 16)] = z16
        return carry

    lax.fori_loop(0, HC, zrow, 0)
    for k in range(640 // HC):
        pltpu.sync_copy(g0, acc.at[pl.ds(s * 640 + k * HC, HC)])
    plsc.subcore_barrier()

    def start_gather(r, h, b):
        idx = cols_v.at[r, pl.ds(h * HC, HC)]

        @pl.when(c == 0)
        def _():
            pltpu.async_copy(ma_hbm.at[idx], GB[b], SG[b])

        @pl.when(c == 1)
        def _():
            pltpu.async_copy(mb_hbm.at[idx], GB[b], SG[b])

    def wait_gather(r, h, b):
        idx = cols_v.at[r, pl.ds(h * HC, HC)]

        @pl.when(c == 0)
        def _():
            pltpu.make_async_copy(ma_hbm.at[idx], GB[b], SG[b]).wait()

        @pl.when(c == 1)
        def _():
            pltpu.make_async_copy(mb_hbm.at[idx], GB[b], SG[b]).wait()

    def start_scatter(r, h, b):
        pltpu.async_copy(GB[b], acc.at[rows_v.at[r, pl.ds(h * HC, HC)]],
                         SS[b], add=True)

    def wait_scatter(r, h, b):
        pltpu.make_async_copy(
            GB[b], acc.at[rows_v.at[r, pl.ds(h * HC, HC)]], SS[b]).wait()

    dnums = lax.GatherDimensionNumbers(
        offset_dims=(), collapsed_slice_dims=(0,), start_index_map=(0,))

    def scale(r, h, b):
        buf = GB[b]

        def kk_body(kk, carry2):
            ev16 = ev_v[r, pl.ds(h * HC + kk * 16, 16)]
            for j in range(16):
                sval = lax.gather(
                    ev16, jnp.full((16, 1), j, jnp.int32), dnums, (1,),
                    mode=lax.GatherScatterMode.PROMISE_IN_BOUNDS)
                e = kk * 16 + j
                for hh in range(H // 16):
                    buf[e, pl.ds(hh * 16, 16)] = (
                        buf[e, pl.ds(hh * 16, 16)] * sval)
            return carry2

        lax.fori_loop(0, HC // 16, kk_body, 0)

    for blk in range(TR_C // BLK_C):
        base = s * TR_C + blk * BLK_C
        pltpu.sync_copy(rows_hbm.at[pl.ds(base, BLK_C)], rows_v)
        pltpu.sync_copy(cols_hbm.at[pl.ds(base, BLK_C)], cols_v)
        pltpu.sync_copy(ev_hbm.at[pl.ds(base, BLK_C)], ev_v)
        start_gather(0, 0, 0)
        start_gather(0, 1, 1)

        def group(j, carry):
            r0 = 3 * j
            for k in range(6):
                r, h, b = r0 + k // 2, k % 2, k % 3
                wait_gather(r, h, b)
                scale(r, h, b)
                start_scatter(r, h, b)
                r2, h2, b2 = r0 + (k + 2) // 2, (k + 2) % 2, (k + 2) % 3
                rm, hm = r0 + (k - 1) // 2, (k - 1) % 2
                if k == 0:
                    @pl.when(j > 0)
                    def _():
                        wait_scatter(rm, hm, b2)
                else:
                    wait_scatter(rm, hm, b2)
                start_gather(r2, h2, b2)
            return carry

        NG = BLK_C // 3
        lax.fori_loop(0, NG, group, 0)

        rl = BLK_C - 1
        wait_gather(rl, 0, 0)
        scale(rl, 0, 0)
        start_scatter(rl, 0, 0)
        wait_gather(rl, 1, 1)
        scale(rl, 1, 1)
        start_scatter(rl, 1, 1)
        wait_scatter(rl - 1, 1, 2)
        wait_scatter(rl, 0, 0)
        wait_scatter(rl, 1, 1)

    plsc.subcore_barrier()

    for k in range(5):
        r0 = s * 640 + k * CH

        @pl.when(r0 + CH <= N)
        def _():
            pltpu.sync_copy(acc.at[pl.ds(r0, CH)],
                            out_hbm.at[pl.ds(r0, CH), pl.ds(c * H, H)])

    @pl.when(s == NS - 1)
    def _():
        r0t = (NS - 1) * 640 + 3 * CH
        pltpu.sync_copy(acc.at[pl.ds(r0t, N - r0t)],
                        out_hbm.at[pl.ds(r0t, N - r0t), pl.ds(c * H, H)])


def _scale_body(a_ref, dis_ref, o_ref):
    o_ref[...] = a_ref[...] * dis_ref[...]


_scale = pl.pallas_call(
    _scale_body,
    grid=(10,),
    in_specs=[
        pl.BlockSpec((N // 10, D), lambda i: (i, 0)),
        pl.BlockSpec((N // 10, 1), lambda i: (i, 0)),
    ],
    out_specs=pl.BlockSpec((N // 10, D), lambda i: (i, 0)),
    out_shape=jax.ShapeDtypeStruct((N, D), jnp.float32),
)


def kernel(x, edge_index, edge_values, W):
    row = edge_index[0].astype(jnp.int32)
    col = edge_index[1].astype(jnp.int32)
    ev = edge_values.astype(jnp.float32)
    rows2d = jnp.pad(row, (0, EP - E)).reshape(EROWS, CH)
    cols2d = jnp.pad(col, (0, EP - E)).reshape(EROWS, CH)
    ev2d = jnp.pad(ev, (0, EP - E)).reshape(EROWS, CH)

    partials = _rowsum(rows2d, ev2d)
    dis = _dis(partials)
    disN = dis[:N].reshape(N, 1)
    ma, mb = _matmul(x, W, disN)
    raw = _spmm(rows2d, cols2d, ev2d, ma, mb)
    return _scale(raw, disN)
